# baseline (speedup 1.0000x reference)
import jax
import jax.numpy as jnp
from jax.experimental import pallas as pl
from jax.experimental.pallas import tpu as pltpu

KTOP = 8
BM = 256


def _nt_dot(a, w):
    return jax.lax.dot_general(a, w, (((1,), (1,)), ((), ())),
                               preferred_element_type=jnp.float32)


def _gate_kernel(xs_ref, g_ref, w1_ref, b1_ref, w2_ref, b2_ref, w3_ref, b3_ref,
                 wout_ref, iout_ref):
    xs = xs_ref[...]
    s1 = xs + pltpu.roll(xs, 511, 1)
    s = s1 + pltpu.roll(s1, 510, 1)
    sb = (s * 0.25).astype(jnp.bfloat16)
    p = jax.lax.dot_general(sb, g_ref[...], (((1,), (0,)), ((), ())),
                            preferred_element_type=jnp.float32)
    pooled = p.astype(jnp.bfloat16).reshape(BM, 2048)

    h1 = jnp.maximum(_nt_dot(pooled, w1_ref[...]) + b1_ref[...], 0.0)
    h1 = h1.astype(jnp.bfloat16)
    h2 = jnp.maximum(_nt_dot(h1, w2_ref[...]) + b2_ref[...], 0.0)
    h2 = h2.astype(jnp.bfloat16)
    logits = _nt_dot(h2, w3_ref[...]) + b3_ref[...]

    z = logits
    iota = jax.lax.broadcasted_iota(jnp.int32, z.shape, 1).astype(jnp.float32)
    vals, idxs = [], []
    for _ in range(KTOP):
        m = jnp.max(z, axis=1, keepdims=True)
        idx = jnp.min(jnp.where(z == m, iota, 64.0), axis=1, keepdims=True)
        vals.append(m)
        idxs.append(idx)
        z = jnp.where(iota == idx, -jnp.inf, z)
    w = jnp.concatenate(vals, axis=1)
    e = jnp.exp(w - w[:, :1])
    wout_ref[...] = e / jnp.sum(e, axis=1, keepdims=True)
    iout_ref[...] = jnp.concatenate(idxs, axis=1).astype(jnp.int32)


@jax.jit
def kernel(x, W1, b1, W2, b2, W3, b3):
    M = x.shape[0]
    xs = x.reshape(M * 16, 512)
    w1 = W1.astype(jnp.bfloat16)
    w2 = W2.astype(jnp.bfloat16)
    w3 = W3.astype(jnp.bfloat16)
    b1r = b1.reshape(1, -1)
    b2r = b2.reshape(1, -1)
    b3r = b3.reshape(1, -1)
    g = jnp.zeros((512, 128), jnp.bfloat16).at[
        4 * jnp.arange(128), jnp.arange(128)].set(jnp.bfloat16(1.0))

    grid = (M // BM,)
    const = lambda i: (0, 0)
    wout, iout = pl.pallas_call(
        _gate_kernel,
        grid=grid,
        in_specs=[
            pl.BlockSpec((BM * 16, 512), lambda i: (i, 0)),
            pl.BlockSpec((512, 128), const),
            pl.BlockSpec((4096, 2048), const),
            pl.BlockSpec((1, 4096), const),
            pl.BlockSpec((2048, 4096), const),
            pl.BlockSpec((1, 2048), const),
            pl.BlockSpec((64, 2048), const),
            pl.BlockSpec((1, 64), const),
        ],
        out_specs=[
            pl.BlockSpec((BM, KTOP), lambda i: (i, 0)),
            pl.BlockSpec((BM, KTOP), lambda i: (i, 0)),
        ],
        out_shape=[
            jax.ShapeDtypeStruct((M, KTOP), jnp.float32),
            jax.ShapeDtypeStruct((M, KTOP), jnp.int32),
        ],
    )(xs, g, w1, b1r, w2, b2r, w3, b3r)
    return (wout, iout)


# speedup vs baseline: 21.4658x; 21.4658x over previous
import jax
import jax.numpy as jnp
from jax.experimental import pallas as pl
from jax.experimental.pallas import tpu as pltpu

KTOP = 8
BM = 256


def _nt_dot(a, w):
    return jax.lax.dot_general(a, w, (((1,), (1,)), ((), ())),
                               preferred_element_type=jnp.float32)


def _gate_kernel(x0_ref, x1_ref, x2_ref, x3_ref, w1_ref, b1_ref, w2_ref,
                 b2_ref, w3_ref, b3_ref, wout_ref, iout_ref):
    pooled = ((x0_ref[...] + x2_ref[...]) +
              (x1_ref[...] + x3_ref[...])) * 0.25
    h0 = pooled.astype(jnp.bfloat16)
    h1 = jnp.maximum(_nt_dot(h0, w1_ref[...]) + b1_ref[...], 0.0)
    h1 = h1.astype(jnp.bfloat16)
    h2 = jnp.maximum(_nt_dot(h1, w2_ref[...]) + b2_ref[...], 0.0)
    h2 = h2.astype(jnp.bfloat16)
    logits = _nt_dot(h2, w3_ref[...]) + b3_ref[...]

    z = logits
    iota = jax.lax.broadcasted_iota(jnp.int32, z.shape, 1).astype(jnp.float32)
    vals, idxs = [], []
    for _ in range(KTOP):
        m = jnp.max(z, axis=1, keepdims=True)
        idx = jnp.min(jnp.where(z == m, iota, 64.0), axis=1, keepdims=True)
        vals.append(m)
        idxs.append(idx)
        z = jnp.where(iota == idx, -jnp.inf, z)
    w = jnp.concatenate(vals, axis=1)
    e = jnp.exp(w - w[:, :1])
    wout_ref[...] = e / jnp.sum(e, axis=1, keepdims=True)
    iout_ref[...] = jnp.concatenate(idxs, axis=1).astype(jnp.int32)


@jax.jit
def kernel(x, W1, b1, W2, b2, W3, b3):
    M = x.shape[0]
    x0 = x[:, :, 0]
    x1 = x[:, :, 1]
    x2 = x[:, :, 2]
    x3 = x[:, :, 3]
    w1 = W1.astype(jnp.bfloat16)
    w2 = W2.astype(jnp.bfloat16)
    w3 = W3.astype(jnp.bfloat16)
    b1r = b1.reshape(1, -1)
    b2r = b2.reshape(1, -1)
    b3r = b3.reshape(1, -1)

    grid = (M // BM,)
    const = lambda i: (0, 0)
    xspec = pl.BlockSpec((BM, 2048), lambda i: (i, 0))
    wout, iout = pl.pallas_call(
        _gate_kernel,
        grid=grid,
        in_specs=[
            xspec, xspec, xspec, xspec,
            pl.BlockSpec((4096, 2048), const),
            pl.BlockSpec((1, 4096), const),
            pl.BlockSpec((2048, 4096), const),
            pl.BlockSpec((1, 2048), const),
            pl.BlockSpec((64, 2048), const),
            pl.BlockSpec((1, 64), const),
        ],
        out_specs=[
            pl.BlockSpec((BM, KTOP), lambda i: (i, 0)),
            pl.BlockSpec((BM, KTOP), lambda i: (i, 0)),
        ],
        out_shape=[
            jax.ShapeDtypeStruct((M, KTOP), jnp.float32),
            jax.ShapeDtypeStruct((M, KTOP), jnp.int32),
        ],
    )(x0, x1, x2, x3, w1, b1r, w2, b2r, w3, b3r)
    return (wout, iout)
